# baseline (speedup 1.0000x reference)
import jax
import jax.numpy as jnp
from jax.experimental import pallas as pl
from jax.experimental.pallas import tpu as pltpu

_NBUF = 8
_LOOKAHEAD = 7
_BM = 512


def _linear_kernel(f_hbm, w_ref, b_ref, o_ref, buf_ref, sem):
    n = f_hbm.shape[0] // _BM

    def copy_for_block(j):
        slot = jax.lax.rem(j, _NBUF)
        return pltpu.make_async_copy(
            f_hbm.at[pl.ds(j * _BM, _BM), :],
            buf_ref.at[slot],
            sem.at[slot],
        )

    for j in range(_LOOKAHEAD):
        copy_for_block(j).start()

    w_bf = w_ref[...].astype(jnp.bfloat16)
    bias = b_ref[...]

    def body(j, carry):
        @pl.when(j + _LOOKAHEAD < n)
        def _issue_ahead():
            copy_for_block(j + _LOOKAHEAD).start()

        copy_for_block(j).wait()
        f = buf_ref[jax.lax.rem(j, _NBUF)]
        acc = jax.lax.dot_general(
            f.astype(jnp.bfloat16), w_bf,
            dimension_numbers=(((1,), (1,)), ((), ())),
            preferred_element_type=jnp.float32,
        )
        o_ref[pl.ds(j * _BM, _BM), :] = acc + bias
        return carry

    jax.lax.fori_loop(0, n, body, 0, unroll=False)


def kernel(feat, W, b):
    B, D = feat.shape
    C = W.shape[0]
    return pl.pallas_call(
        _linear_kernel,
        in_specs=[
            pl.BlockSpec(memory_space=pltpu.MemorySpace.HBM),
            pl.BlockSpec((C, D), lambda: (0, 0)),
            pl.BlockSpec((1, C), lambda: (0, 0)),
        ],
        out_specs=pl.BlockSpec((B, C), lambda: (0, 0)),
        out_shape=jax.ShapeDtypeStruct((B, C), jnp.float32),
        scratch_shapes=[
            pltpu.VMEM((_NBUF, _BM, D), jnp.float32),
            pltpu.SemaphoreType.DMA((_NBUF,)),
        ],
    )(feat, W, b.reshape(1, C))


# speedup vs baseline: 1.0609x; 1.0609x over previous
import jax
import jax.numpy as jnp
from jax.experimental import pallas as pl
from jax.experimental.pallas import tpu as pltpu


def _linear_kernel(f_ref, w_ref, b_ref, o_ref):
    acc = jax.lax.dot_general(
        f_ref[...].astype(jnp.bfloat16), w_ref[...].astype(jnp.bfloat16),
        dimension_numbers=(((1,), (1,)), ((), ())),
        preferred_element_type=jnp.float32,
    )
    o_ref[...] = acc + b_ref[...]


def kernel(feat, W, b):
    B, D = feat.shape
    C = W.shape[0]
    Bm = 1024
    return pl.pallas_call(
        _linear_kernel,
        grid=(B // Bm,),
        in_specs=[
            pl.BlockSpec((Bm, D), lambda i: (i, 0)),
            pl.BlockSpec((C, D), lambda i: (0, 0)),
            pl.BlockSpec((1, C), lambda i: (0, 0)),
        ],
        out_specs=pl.BlockSpec((Bm, C), lambda i: (i, 0)),
        out_shape=jax.ShapeDtypeStruct((B, C), jnp.float32),
        compiler_params=pltpu.CompilerParams(
            dimension_semantics=("arbitrary",),
            skip_device_barrier=True,
        ),
    )(feat, W, b.reshape(1, C))
